# baseline (speedup 1.0000x reference)
import functools

import jax
import jax.numpy as jnp
import numpy as np
from jax.experimental import pallas as pl
from jax.experimental.pallas import tpu as pltpu

B, S, D = 1, 2048, 1024
H, KV, HD = 16, 4, 64
E, K, I = 8, 2, 512
EPS = 1e-6
THETA = 1000000.0
HALF = HD // 2
N_REP = H // KV
SCALE = 1.0 / 8.0
VA = 2 * HD

TS1 = 1024
TSQ = 512
TS3 = 512
TSM = 1024


def _rms(x, w, eps=EPS):
    nrm = jax.lax.rsqrt(jnp.mean(jnp.square(x), axis=-1, keepdims=True) + eps)
    return x * nrm * w


def _swap_halves(x):
    n = x.shape[-1]
    lane = jax.lax.broadcasted_iota(jnp.int32, x.shape, 1)
    lo = (lane % HD) < HALF
    return jnp.where(lo, pltpu.roll(x, n - HALF, 1), pltpu.roll(x, HALF, 1))


def _norm_rope(x, bsum_ref, bbc_ref, t1_ref, t2_ref, w1_ref, w2_ref):
    sq = x * x
    ms = jnp.dot(sq, bsum_ref[...], preferred_element_type=jnp.float32)
    r = jax.lax.rsqrt(ms + EPS)
    rb = jnp.dot(r, bbc_ref[...], preferred_element_type=jnp.float32)
    return rb * (x * (t1_ref[...] * w1_ref[...])
                 + _swap_halves(x) * (t2_ref[...] * w2_ref[...]))


def _qkv_kernel(x_ref, nw_ref, wq_ref, wk_ref, wv_ref,
                bsq_ref, bbq_ref, t1q_ref, t2q_ref, w1q_ref, w2q_ref,
                bsk_ref, bbk_ref, t1k_ref, t2k_ref, w1k_ref, w2k_ref,
                q_ref, k_ref, v_ref):
    h = _rms(x_ref[...], nw_ref[...]).astype(jnp.bfloat16)
    q = jnp.dot(h, wq_ref[...].astype(jnp.bfloat16),
                preferred_element_type=jnp.float32)
    k = jnp.dot(h, wk_ref[...].astype(jnp.bfloat16),
                preferred_element_type=jnp.float32)
    v = jnp.dot(h, wv_ref[...].astype(jnp.bfloat16),
                preferred_element_type=jnp.float32)
    q_ref[...] = _norm_rope(q, bsq_ref, bbq_ref, t1q_ref, t2q_ref,
                            w1q_ref, w2q_ref).astype(jnp.bfloat16)
    kr = _norm_rope(k, bsk_ref, bbk_ref, t1k_ref, t2k_ref,
                    w1k_ref, w2k_ref).astype(jnp.bfloat16)
    lane = jax.lax.broadcasted_iota(jnp.int32, (TS1, HD), 1)
    onescol = jnp.where(lane == 0, 1.0, 0.0).astype(jnp.bfloat16)
    for g in range(KV):
        k_ref[g] = kr[:, g * HD:(g + 1) * HD]
        vg = v[:, g * HD:(g + 1) * HD].astype(jnp.bfloat16)
        v_ref[g] = jnp.concatenate([vg, onescol], axis=1)


def _attn_kernel(q_ref, k_ref, v_ref, o_ref):
    row = jax.lax.broadcasted_iota(jnp.int32, (TSQ, TSQ), 0)
    col = jax.lax.broadcasted_iota(jnp.int32, (TSQ, TSQ), 1)
    tri = col <= row
    for qi in range(S // TSQ):
        q2 = q_ref[qi * TSQ:(qi + 1) * TSQ, :]
        qa = q2[:, :HD]
        qb = q2[:, HD:]
        acc_a = jnp.zeros((TSQ, VA), jnp.float32)
        acc_b = jnp.zeros((TSQ, VA), jnp.float32)
        for kj in range(qi + 1):
            kb = k_ref[0, kj * TSQ:(kj + 1) * TSQ, :]
            vb = v_ref[0, kj * TSQ:(kj + 1) * TSQ, :]
            sa = jax.lax.dot_general(qa, kb, (((1,), (1,)), ((), ())),
                                     preferred_element_type=jnp.float32)
            sb = jax.lax.dot_general(qb, kb, (((1,), (1,)), ((), ())),
                                     preferred_element_type=jnp.float32)
            ea = jnp.exp(sa - 8.0)
            eb = jnp.exp(sb - 8.0)
            if kj == qi:
                ea = jnp.where(tri, ea, 0.0)
                eb = jnp.where(tri, eb, 0.0)
            acc_a = acc_a + jnp.dot(ea.astype(jnp.bfloat16), vb,
                                    preferred_element_type=jnp.float32)
            acc_b = acc_b + jnp.dot(eb.astype(jnp.bfloat16), vb,
                                    preferred_element_type=jnp.float32)
        oa = acc_a[:, :HD] * (1.0 / acc_a[:, HD:HD + 1])
        ob = acc_b[:, :HD] * (1.0 / acc_b[:, HD:HD + 1])
        o_ref[qi * TSQ:(qi + 1) * TSQ, :] = jnp.concatenate(
            [oa, ob], axis=1).astype(jnp.bfloat16)


def _moe_kernel(ao_ref, wo_ref, x_ref, fw_ref, wg_ref,
                wgt_ref, wup_ref, wdn_ref, o_ref, hts_ref, ws_ref):
    e = pl.program_id(1)

    @pl.when(e == 0)
    def _():
        h2 = x_ref[...] + jnp.dot(ao_ref[...], wo_ref[...].astype(jnp.bfloat16),
                                  preferred_element_type=jnp.float32)
        o_ref[...] = h2
        ht = _rms(h2, fw_ref[...])
        hts_ref[...] = ht.astype(jnp.bfloat16)
        logits = jnp.dot(ht.astype(jnp.bfloat16),
                         wg_ref[...].astype(jnp.bfloat16),
                         preferred_element_type=jnp.float32)
        m = jnp.max(logits, axis=-1, keepdims=True)
        eg = jnp.exp(logits - m)
        gates = eg / jnp.sum(eg, axis=-1, keepdims=True)
        lane = jax.lax.broadcasted_iota(jnp.int32, gates.shape, 1)
        a1 = jnp.argmax(gates, axis=-1)[:, None]
        one1 = lane == a1
        v1 = jnp.max(gates, axis=-1, keepdims=True)
        g2 = jnp.where(one1, jnp.float32(-1.0), gates)
        a2 = jnp.argmax(g2, axis=-1)[:, None]
        one2 = lane == a2
        v2 = jnp.max(g2, axis=-1, keepdims=True)
        denom = jnp.maximum(v1 + v2, 1e-9)
        ws_ref[...] = (jnp.where(one1, v1, 0.0)
                       + jnp.where(one2, v2, 0.0)) / denom

    ht = hts_ref[...]
    g = jnp.dot(ht, wgt_ref[0].astype(jnp.bfloat16),
                preferred_element_type=jnp.float32)
    u = jnp.dot(ht, wup_ref[0].astype(jnp.bfloat16),
                preferred_element_type=jnp.float32)
    inter = (g * jax.lax.logistic(g)) * u
    eo = jnp.dot(inter.astype(jnp.bfloat16), wdn_ref[0].astype(jnp.bfloat16),
                 preferred_element_type=jnp.float32)
    lane = jax.lax.broadcasted_iota(jnp.int32, ws_ref.shape, 1)
    wcol = jnp.sum(jnp.where(lane == e, ws_ref[...], 0.0), axis=-1,
                   keepdims=True)
    o_ref[...] = o_ref[...] + wcol * eo


def _rope_consts_np(nheads, scale):
    w = nheads * HD
    freqs = 1.0 / (THETA ** (np.arange(0, HD, 2, dtype=np.float32) / HD))
    t = np.arange(S, dtype=np.float32)
    lane = np.arange(w)
    fl = freqs[(lane % HD) % HALF]
    ang = np.outer(t, fl).astype(np.float32)
    lo = (lane % HD) < HALF
    t1 = (np.cos(ang) * scale).astype(np.float32)
    t2 = (np.sin(ang) * np.where(lo, -1.0, 1.0)[None, :] * scale).astype(np.float32)
    grp = lane // HD
    bsum = ((np.arange(nheads)[None, :] == grp[:, None]).astype(np.float32) / HD)
    bbc = (np.arange(nheads)[:, None] == grp[None, :]).astype(np.float32)
    return t1, t2, bsum, bbc


_T1Q, _T2Q, _BSQ, _BBQ = _rope_consts_np(H, SCALE)
_T1K, _T2K, _BSK, _BBK = _rope_consts_np(KV, 1.0)


@functools.partial(jax.jit, static_argnames=())
def kernel(hidden, attn_norm_w, q_norm_w, k_norm_w, ffn_norm_w, Wq, Wk, Wv,
           Wo, Wg, We_gate, We_up, We_down):
    x = hidden.reshape(S, D)
    w1q = jnp.tile(q_norm_w, H).reshape(1, H * HD)
    w2q = jnp.tile(jnp.concatenate([q_norm_w[HALF:], q_norm_w[:HALF]]),
                   H).reshape(1, H * HD)
    w1k = jnp.tile(k_norm_w, KV).reshape(1, KV * HD)
    w2k = jnp.tile(jnp.concatenate([k_norm_w[HALF:], k_norm_w[:HALF]]),
                   KV).reshape(1, KV * HD)

    qh, kh, vh = pl.pallas_call(
        _qkv_kernel,
        grid=(S // TS1,),
        in_specs=[
            pl.BlockSpec((TS1, D), lambda i: (i, 0)),
            pl.BlockSpec((1, D), lambda i: (0, 0)),
            pl.BlockSpec((D, H * HD), lambda i: (0, 0)),
            pl.BlockSpec((D, KV * HD), lambda i: (0, 0)),
            pl.BlockSpec((D, KV * HD), lambda i: (0, 0)),
            pl.BlockSpec((H * HD, H), lambda i: (0, 0)),
            pl.BlockSpec((H, H * HD), lambda i: (0, 0)),
            pl.BlockSpec((TS1, H * HD), lambda i: (i, 0)),
            pl.BlockSpec((TS1, H * HD), lambda i: (i, 0)),
            pl.BlockSpec((1, H * HD), lambda i: (0, 0)),
            pl.BlockSpec((1, H * HD), lambda i: (0, 0)),
            pl.BlockSpec((KV * HD, KV), lambda i: (0, 0)),
            pl.BlockSpec((KV, KV * HD), lambda i: (0, 0)),
            pl.BlockSpec((TS1, KV * HD), lambda i: (i, 0)),
            pl.BlockSpec((TS1, KV * HD), lambda i: (i, 0)),
            pl.BlockSpec((1, KV * HD), lambda i: (0, 0)),
            pl.BlockSpec((1, KV * HD), lambda i: (0, 0)),
        ],
        out_specs=[
            pl.BlockSpec((TS1, H * HD), lambda i: (i, 0)),
            pl.BlockSpec((KV, TS1, HD), lambda i: (0, i, 0)),
            pl.BlockSpec((KV, TS1, VA), lambda i: (0, i, 0)),
        ],
        out_shape=[
            jax.ShapeDtypeStruct((S, H * HD), jnp.bfloat16),
            jax.ShapeDtypeStruct((KV, S, HD), jnp.bfloat16),
            jax.ShapeDtypeStruct((KV, S, VA), jnp.bfloat16),
        ],
    )(x, attn_norm_w.reshape(1, D), Wq, Wk, Wv,
      _BSQ, _BBQ, _T1Q, _T2Q, w1q, w2q,
      _BSK, _BBK, _T1K, _T2K, w1k, w2k)

    attn_out = pl.pallas_call(
        _attn_kernel,
        grid=(KV, N_REP // 2),
        in_specs=[
            pl.BlockSpec((S, 2 * HD), lambda g, p: (0, 2 * g + p)),
            pl.BlockSpec((1, S, HD), lambda g, p: (g, 0, 0)),
            pl.BlockSpec((1, S, VA), lambda g, p: (g, 0, 0)),
        ],
        out_specs=pl.BlockSpec((S, 2 * HD), lambda g, p: (0, 2 * g + p)),
        out_shape=jax.ShapeDtypeStruct((S, H * HD), jnp.bfloat16),
    )(qh, kh, vh)

    out = pl.pallas_call(
        _moe_kernel,
        grid=(S // TSM, E),
        in_specs=[
            pl.BlockSpec((TSM, H * HD), lambda t, e: (t, 0)),
            pl.BlockSpec((H * HD, D), lambda t, e: (0, 0)),
            pl.BlockSpec((TSM, D), lambda t, e: (t, 0)),
            pl.BlockSpec((1, D), lambda t, e: (0, 0)),
            pl.BlockSpec((D, E), lambda t, e: (0, 0)),
            pl.BlockSpec((1, D, I), lambda t, e: (e, 0, 0)),
            pl.BlockSpec((1, D, I), lambda t, e: (e, 0, 0)),
            pl.BlockSpec((1, I, D), lambda t, e: (e, 0, 0)),
        ],
        out_specs=pl.BlockSpec((TSM, D), lambda t, e: (t, 0)),
        out_shape=jax.ShapeDtypeStruct((S, D), jnp.float32),
        scratch_shapes=[pltpu.VMEM((TSM, D), jnp.bfloat16),
                        pltpu.VMEM((TSM, E), jnp.float32)],
    )(attn_out, Wo, x, ffn_norm_w.reshape(1, D), Wg,
      We_gate, We_up, We_down)

    return out.reshape(B, S, D)


# speedup vs baseline: 1.0172x; 1.0172x over previous
import functools

import jax
import jax.numpy as jnp
import numpy as np
from jax.experimental import pallas as pl
from jax.experimental.pallas import tpu as pltpu

B, S, D = 1, 2048, 1024
H, KV, HD = 16, 4, 64
E, K, I = 8, 2, 512
EPS = 1e-6
THETA = 1000000.0
HALF = HD // 2
N_REP = H // KV
SCALE = 1.0 / 8.0
VA = 2 * HD

TS1 = 512
TSQ = 512
TS3 = 512
TSM = 1024


def _rms(x, w, eps=EPS):
    nrm = jax.lax.rsqrt(jnp.mean(jnp.square(x), axis=-1, keepdims=True) + eps)
    return x * nrm * w


def _swap_halves(x):
    n = x.shape[-1]
    lane = jax.lax.broadcasted_iota(jnp.int32, x.shape, 1)
    lo = (lane % HD) < HALF
    return jnp.where(lo, pltpu.roll(x, n - HALF, 1), pltpu.roll(x, HALF, 1))


def _norm_rope(x, bsum_ref, bbc_ref, t1_ref, t2_ref, w1_ref, w2_ref):
    sq = x * x
    ms = jnp.dot(sq, bsum_ref[...], preferred_element_type=jnp.float32)
    r = jax.lax.rsqrt(ms + EPS)
    rb = jnp.dot(r, bbc_ref[...], preferred_element_type=jnp.float32)
    return rb * (x * (t1_ref[...] * w1_ref[...])
                 + _swap_halves(x) * (t2_ref[...] * w2_ref[...]))


def _qkv_kernel(x_ref, nw_ref, wq_ref, wk_ref, wv_ref,
                bsq_ref, bbq_ref, t1q_ref, t2q_ref, w1q_ref, w2q_ref,
                bsk_ref, bbk_ref, t1k_ref, t2k_ref, w1k_ref, w2k_ref,
                q_ref, k_ref, v_ref):
    h = _rms(x_ref[...], nw_ref[...]).astype(jnp.bfloat16)
    q = jnp.dot(h, wq_ref[...].astype(jnp.bfloat16),
                preferred_element_type=jnp.float32)
    k = jnp.dot(h, wk_ref[...].astype(jnp.bfloat16),
                preferred_element_type=jnp.float32)
    v = jnp.dot(h, wv_ref[...].astype(jnp.bfloat16),
                preferred_element_type=jnp.float32)
    q_ref[...] = _norm_rope(q, bsq_ref, bbq_ref, t1q_ref, t2q_ref,
                            w1q_ref, w2q_ref).astype(jnp.bfloat16)
    kr = _norm_rope(k, bsk_ref, bbk_ref, t1k_ref, t2k_ref,
                    w1k_ref, w2k_ref).astype(jnp.bfloat16)
    lane = jax.lax.broadcasted_iota(jnp.int32, (TS1, HD), 1)
    onescol = jnp.where(lane == 0, 1.0, 0.0).astype(jnp.bfloat16)
    for g in range(KV):
        k_ref[g] = kr[:, g * HD:(g + 1) * HD]
        vg = v[:, g * HD:(g + 1) * HD].astype(jnp.bfloat16)
        v_ref[g] = jnp.concatenate([vg, onescol], axis=1)


def _attn_kernel(q_ref, k_ref, v_ref, o_ref):
    row = jax.lax.broadcasted_iota(jnp.int32, (TSQ, TSQ), 0)
    col = jax.lax.broadcasted_iota(jnp.int32, (TSQ, TSQ), 1)
    tri = col <= row
    for qi in range(S // TSQ):
        q2 = q_ref[qi * TSQ:(qi + 1) * TSQ, :]
        qa = q2[:, :HD]
        qb = q2[:, HD:]
        acc_a = jnp.zeros((TSQ, VA), jnp.float32)
        acc_b = jnp.zeros((TSQ, VA), jnp.float32)
        for kj in range(qi + 1):
            kb = k_ref[0, kj * TSQ:(kj + 1) * TSQ, :]
            vb = v_ref[0, kj * TSQ:(kj + 1) * TSQ, :]
            sa = jax.lax.dot_general(qa, kb, (((1,), (1,)), ((), ())),
                                     preferred_element_type=jnp.float32)
            sb = jax.lax.dot_general(qb, kb, (((1,), (1,)), ((), ())),
                                     preferred_element_type=jnp.float32)
            ea = jnp.exp(sa - 8.0)
            eb = jnp.exp(sb - 8.0)
            if kj == qi:
                ea = jnp.where(tri, ea, 0.0)
                eb = jnp.where(tri, eb, 0.0)
            acc_a = acc_a + jnp.dot(ea.astype(jnp.bfloat16), vb,
                                    preferred_element_type=jnp.float32)
            acc_b = acc_b + jnp.dot(eb.astype(jnp.bfloat16), vb,
                                    preferred_element_type=jnp.float32)
        oa = acc_a[:, :HD] * (1.0 / acc_a[:, HD:HD + 1])
        ob = acc_b[:, :HD] * (1.0 / acc_b[:, HD:HD + 1])
        o_ref[qi * TSQ:(qi + 1) * TSQ, :] = jnp.concatenate(
            [oa, ob], axis=1).astype(jnp.bfloat16)


def _moe_kernel(ao_ref, wo_ref, x_ref, fw_ref, wg_ref,
                wgt_ref, wup_ref, wdn_ref, o_ref, hts_ref, ws_ref):
    e = pl.program_id(1)

    @pl.when(e == 0)
    def _():
        h2 = x_ref[...] + jnp.dot(ao_ref[...], wo_ref[...].astype(jnp.bfloat16),
                                  preferred_element_type=jnp.float32)
        o_ref[...] = h2
        ht = _rms(h2, fw_ref[...])
        hts_ref[...] = ht.astype(jnp.bfloat16)
        logits = jnp.dot(ht.astype(jnp.bfloat16),
                         wg_ref[...].astype(jnp.bfloat16),
                         preferred_element_type=jnp.float32)
        m = jnp.max(logits, axis=-1, keepdims=True)
        eg = jnp.exp(logits - m)
        gates = eg / jnp.sum(eg, axis=-1, keepdims=True)
        lane = jax.lax.broadcasted_iota(jnp.int32, gates.shape, 1)
        a1 = jnp.argmax(gates, axis=-1)[:, None]
        one1 = lane == a1
        v1 = jnp.max(gates, axis=-1, keepdims=True)
        g2 = jnp.where(one1, jnp.float32(-1.0), gates)
        a2 = jnp.argmax(g2, axis=-1)[:, None]
        one2 = lane == a2
        v2 = jnp.max(g2, axis=-1, keepdims=True)
        denom = jnp.maximum(v1 + v2, 1e-9)
        ws_ref[...] = (jnp.where(one1, v1, 0.0)
                       + jnp.where(one2, v2, 0.0)) / denom

    ht = hts_ref[...]
    g = jnp.dot(ht, wgt_ref[0].astype(jnp.bfloat16),
                preferred_element_type=jnp.float32)
    u = jnp.dot(ht, wup_ref[0].astype(jnp.bfloat16),
                preferred_element_type=jnp.float32)
    inter = (g * jax.lax.logistic(g)) * u
    eo = jnp.dot(inter.astype(jnp.bfloat16), wdn_ref[0].astype(jnp.bfloat16),
                 preferred_element_type=jnp.float32)
    lane = jax.lax.broadcasted_iota(jnp.int32, ws_ref.shape, 1)
    wcol = jnp.sum(jnp.where(lane == e, ws_ref[...], 0.0), axis=-1,
                   keepdims=True)
    o_ref[...] = o_ref[...] + wcol * eo


def _rope_consts_np(nheads, scale):
    w = nheads * HD
    freqs = 1.0 / (THETA ** (np.arange(0, HD, 2, dtype=np.float32) / HD))
    t = np.arange(S, dtype=np.float32)
    lane = np.arange(w)
    fl = freqs[(lane % HD) % HALF]
    ang = np.outer(t, fl).astype(np.float32)
    lo = (lane % HD) < HALF
    t1 = (np.cos(ang) * scale).astype(np.float32)
    t2 = (np.sin(ang) * np.where(lo, -1.0, 1.0)[None, :] * scale).astype(np.float32)
    grp = lane // HD
    bsum = ((np.arange(nheads)[None, :] == grp[:, None]).astype(np.float32) / HD)
    bbc = (np.arange(nheads)[:, None] == grp[None, :]).astype(np.float32)
    return t1, t2, bsum, bbc


_T1Q, _T2Q, _BSQ, _BBQ = _rope_consts_np(H, SCALE)
_T1K, _T2K, _BSK, _BBK = _rope_consts_np(KV, 1.0)


@functools.partial(jax.jit, static_argnames=())
def kernel(hidden, attn_norm_w, q_norm_w, k_norm_w, ffn_norm_w, Wq, Wk, Wv,
           Wo, Wg, We_gate, We_up, We_down):
    x = hidden.reshape(S, D)
    w1q = jnp.tile(q_norm_w, H).reshape(1, H * HD)
    w2q = jnp.tile(jnp.concatenate([q_norm_w[HALF:], q_norm_w[:HALF]]),
                   H).reshape(1, H * HD)
    w1k = jnp.tile(k_norm_w, KV).reshape(1, KV * HD)
    w2k = jnp.tile(jnp.concatenate([k_norm_w[HALF:], k_norm_w[:HALF]]),
                   KV).reshape(1, KV * HD)

    qh, kh, vh = pl.pallas_call(
        _qkv_kernel,
        grid=(S // TS1,),
        in_specs=[
            pl.BlockSpec((TS1, D), lambda i: (i, 0)),
            pl.BlockSpec((1, D), lambda i: (0, 0)),
            pl.BlockSpec((D, H * HD), lambda i: (0, 0)),
            pl.BlockSpec((D, KV * HD), lambda i: (0, 0)),
            pl.BlockSpec((D, KV * HD), lambda i: (0, 0)),
            pl.BlockSpec((H * HD, H), lambda i: (0, 0)),
            pl.BlockSpec((H, H * HD), lambda i: (0, 0)),
            pl.BlockSpec((TS1, H * HD), lambda i: (i, 0)),
            pl.BlockSpec((TS1, H * HD), lambda i: (i, 0)),
            pl.BlockSpec((1, H * HD), lambda i: (0, 0)),
            pl.BlockSpec((1, H * HD), lambda i: (0, 0)),
            pl.BlockSpec((KV * HD, KV), lambda i: (0, 0)),
            pl.BlockSpec((KV, KV * HD), lambda i: (0, 0)),
            pl.BlockSpec((TS1, KV * HD), lambda i: (i, 0)),
            pl.BlockSpec((TS1, KV * HD), lambda i: (i, 0)),
            pl.BlockSpec((1, KV * HD), lambda i: (0, 0)),
            pl.BlockSpec((1, KV * HD), lambda i: (0, 0)),
        ],
        out_specs=[
            pl.BlockSpec((TS1, H * HD), lambda i: (i, 0)),
            pl.BlockSpec((KV, TS1, HD), lambda i: (0, i, 0)),
            pl.BlockSpec((KV, TS1, VA), lambda i: (0, i, 0)),
        ],
        out_shape=[
            jax.ShapeDtypeStruct((S, H * HD), jnp.bfloat16),
            jax.ShapeDtypeStruct((KV, S, HD), jnp.bfloat16),
            jax.ShapeDtypeStruct((KV, S, VA), jnp.bfloat16),
        ],
    )(x, attn_norm_w.reshape(1, D), Wq, Wk, Wv,
      _BSQ, _BBQ, _T1Q, _T2Q, w1q, w2q,
      _BSK, _BBK, _T1K, _T2K, w1k, w2k)

    attn_out = pl.pallas_call(
        _attn_kernel,
        grid=(KV, N_REP // 2),
        in_specs=[
            pl.BlockSpec((S, 2 * HD), lambda g, p: (0, 2 * g + p)),
            pl.BlockSpec((1, S, HD), lambda g, p: (g, 0, 0)),
            pl.BlockSpec((1, S, VA), lambda g, p: (g, 0, 0)),
        ],
        out_specs=pl.BlockSpec((S, 2 * HD), lambda g, p: (0, 2 * g + p)),
        out_shape=jax.ShapeDtypeStruct((S, H * HD), jnp.bfloat16),
    )(qh, kh, vh)

    out = pl.pallas_call(
        _moe_kernel,
        grid=(S // TSM, E),
        in_specs=[
            pl.BlockSpec((TSM, H * HD), lambda t, e: (t, 0)),
            pl.BlockSpec((H * HD, D), lambda t, e: (0, 0)),
            pl.BlockSpec((TSM, D), lambda t, e: (t, 0)),
            pl.BlockSpec((1, D), lambda t, e: (0, 0)),
            pl.BlockSpec((D, E), lambda t, e: (0, 0)),
            pl.BlockSpec((1, D, I), lambda t, e: (e, 0, 0)),
            pl.BlockSpec((1, D, I), lambda t, e: (e, 0, 0)),
            pl.BlockSpec((1, I, D), lambda t, e: (e, 0, 0)),
        ],
        out_specs=pl.BlockSpec((TSM, D), lambda t, e: (t, 0)),
        out_shape=jax.ShapeDtypeStruct((S, D), jnp.float32),
        scratch_shapes=[pltpu.VMEM((TSM, D), jnp.bfloat16),
                        pltpu.VMEM((TSM, E), jnp.float32)],
    )(attn_out, Wo, x, ffn_norm_w.reshape(1, D), Wg,
      We_gate, We_up, We_down)

    return out.reshape(B, S, D)
